# Initial kernel scaffold; baseline (speedup 1.0000x reference)
import jax
import jax.numpy as jnp
from jax.experimental import pallas as pl


def kernel(x, edge_index, edge_attr, W1, b1, We1, att1, bias1, W2, b2, We2, att2, bias2):
    raise NotImplementedError("write your pallas kernel here")


# speedup vs baseline: 9.3785x; 9.3785x over previous
import functools

import jax
import jax.numpy as jnp
from jax import lax
from jax.experimental import pallas as pl
from jax.experimental.pallas import tpu as pltpu
from jax.experimental.pallas import tpu_sc as plsc

N = 10000
E = 320000
D1 = 128
D2 = 16
D1P = D1 + 16
D2P = D2 + 16
NC, NS, L = 2, 16, 16
NW = NC * NS


def _lin1_body(x_ref, w_ref, b_ref, o_ref):
    rb = x_ref.shape[0]
    feat = (
        jnp.dot(x_ref[...], w_ref[...], preferred_element_type=jnp.float32)
        + b_ref[...]
    )
    col = lax.broadcasted_iota(jnp.int32, (rb, D1P - D1), 1)
    pad = jnp.where(col == 0, 1.0, 0.0)
    o_ref[...] = jnp.concatenate([feat, pad], axis=1)


def _edge_proj_body(ea_ref, we1_ref, we2_ref, o1_ref, o2_ref):
    ea = ea_ref[...]
    o1_ref[...] = jnp.dot(ea, we1_ref[...], preferred_element_type=jnp.float32)
    o2_ref[...] = jnp.dot(ea, we2_ref[...], preferred_element_type=jnp.float32)


def _norm_lin2_body(acc_ref, bias1_ref, w2_ref, b2_ref, o_ref):
    rb = o_ref.shape[0]
    acc = acc_ref[0] + acc_ref[1]
    den = acc[:, D1] + 1e-16
    lat = acc[:, :D1] / den[:, None] + bias1_ref[...]
    feat = (
        jnp.dot(lat, w2_ref[...], preferred_element_type=jnp.float32)
        + b2_ref[...]
    )
    col = lax.broadcasted_iota(jnp.int32, (rb, D2P - D2), 1)
    pad = jnp.where(col == 0, 1.0, 0.0)
    o_ref[...] = jnp.concatenate([feat, pad], axis=1)


def _norm2_body(acc_ref, bias2_ref, o_ref):
    acc = acc_ref[0] + acc_ref[1]
    den = acc[:, D2] + 1e-16
    o_ref[...] = acc[:, :D2] / den[:, None] + bias2_ref[...]


_MESH = plsc.VectorSubcoreMesh(core_axis_name="c", subcore_axis_name="s")


def _sc_edge_body(dp, dfull, ch, xl_hbm, src_hbm, dst_hbm, ef_hbm, att_hbm,
                  acc_out, acc_sh, srcv, dstv, ab, bb, efb, attv,
                  sem_a, sem_b, sem_e, sem_s):
    cid = lax.axis_index("c")
    sid = lax.axis_index("s")
    wid = sid * NC + cid
    nj = dfull // L
    njp = dp // L
    nchunks = E // ch

    zero16 = jnp.zeros((L,), jnp.float32)

    def _zrow(i, carry):
        for j in range(njp):
            ab[i, pl.ds(L * j, L)] = zero16
        return carry

    lax.fori_loop(0, ch, _zrow, 0)

    base_row = sid * 624
    _sizes, _offs, _rem = [], [], 624
    while _rem:
        _offs.append(624 - _rem)
        _sizes.append(min(_rem, ch))
        _rem -= _sizes[-1]
    for sz, off in zip(_sizes, _offs):
        pltpu.sync_copy(ab.at[pl.ds(0, sz)],
                        acc_sh.at[pl.ds(base_row + off, sz)])

    @pl.when(sid == NS - 1)
    def _():
        pltpu.sync_copy(ab.at[pl.ds(0, 16)], acc_sh.at[pl.ds(624 * NS, 16)])

    plsc.subcore_barrier()

    pltpu.sync_copy(att_hbm, attv)
    att_regs = [attv[pl.ds(L * j, L)] for j in range(nj)]
    lane = lax.iota(jnp.int32, L)

    nch = jnp.where(wid < (nchunks % NW), nchunks // NW + 1, nchunks // NW)

    def _chunk(i, carry):
        k = wid + NW * i
        ebase = k * ch
        pltpu.sync_copy(src_hbm.at[pl.ds(ebase, ch)], srcv)
        pltpu.sync_copy(dst_hbm.at[pl.ds(ebase, ch)], dstv)
        cp_a = pltpu.async_copy(xl_hbm.at[srcv], ab, sem_a)
        cp_b = pltpu.async_copy(xl_hbm.at[dstv], bb, sem_b)
        cp_e = pltpu.async_copy(ef_hbm.at[pl.ds(ebase, ch)], efb, sem_e)
        cp_a.wait()
        cp_b.wait()
        cp_e.wait()

        def _group(g, cc):
            e0 = g * L
            lvec = zero16
            for t in range(L):
                e = e0 + t
                s = zero16
                for j in range(nj):
                    sl = pl.ds(L * j, L)
                    v = ab[e, sl] + bb[e, sl] + efb[e, sl]
                    lv = jnp.where(v > 0, v, 0.2 * v)
                    s = s + lv * att_regs[j]
                lvec = jnp.where(lane == t, jnp.sum(s), lvec)
            wv = jnp.exp(lvec)
            for t in range(L):
                e = e0 + t
                w1 = jnp.sum(jnp.where(lane == t, wv, 0.0))
                for j in range(nj + 1):
                    sl = pl.ds(L * j, L)
                    ab[e, sl] = ab[e, sl] * w1
            return cc

        lax.fori_loop(0, ch // L, _group, 0)
        pltpu.async_copy(ab, acc_sh.at[dstv], sem_s, add=True).wait()
        return carry

    lax.fori_loop(0, nch, _chunk, 0)
    plsc.subcore_barrier()

    for sz, off in zip(_sizes, _offs):
        sl = pl.ds(base_row + off, sz)
        pltpu.sync_copy(acc_sh.at[sl], acc_out.at[cid, sl])

    @pl.when(sid == NS - 1)
    def _():
        sl = pl.ds(624 * NS, 16)
        pltpu.sync_copy(acc_sh.at[sl], acc_out.at[cid, sl])


def _make_sc_edge(dp, dfull, ch):
    return functools.partial(
        pl.kernel,
        out_type=jax.ShapeDtypeStruct((NC, N, dp), jnp.float32),
        mesh=_MESH,
        scratch_types=[
            pltpu.VMEM_SHARED((N, dp), jnp.float32),
            pltpu.VMEM((ch,), jnp.int32),
            pltpu.VMEM((ch,), jnp.int32),
            pltpu.VMEM((ch, dp), jnp.float32),
            pltpu.VMEM((ch, dp), jnp.float32),
            pltpu.VMEM((ch, dfull), jnp.float32),
            pltpu.VMEM((dfull,), jnp.float32),
            pltpu.SemaphoreType.DMA,
            pltpu.SemaphoreType.DMA,
            pltpu.SemaphoreType.DMA,
            pltpu.SemaphoreType.DMA,
        ],
        compiler_params=pltpu.CompilerParams(use_tc_tiling_on_sc=False, needs_layout_passes=False),
    )(functools.partial(_sc_edge_body, dp, dfull, ch))


_sc_edge1 = _make_sc_edge(D1P, D1, 64)
_sc_edge2 = _make_sc_edge(D2P, D2, 128)


def kernel(x, edge_index, edge_attr, W1, b1, We1, att1, bias1,
           W2, b2, We2, att2, bias2):
    src = edge_index[0]
    dst = edge_index[1]

    RB = 400
    xl1 = pl.pallas_call(
        _lin1_body,
        grid=(N // RB,),
        in_specs=[
            pl.BlockSpec((RB, D1), lambda i: (i, 0)),
            pl.BlockSpec((D1, D1), lambda i: (0, 0)),
            pl.BlockSpec((D1,), lambda i: (0,)),
        ],
        out_specs=pl.BlockSpec((RB, D1P), lambda i: (i, 0)),
        out_shape=jax.ShapeDtypeStruct((N, D1P), jnp.float32),
    )(x, W1, b1)

    EB = 2000
    ef1, ef2 = pl.pallas_call(
        _edge_proj_body,
        grid=(E // EB,),
        in_specs=[
            pl.BlockSpec((EB, D2), lambda i: (i, 0)),
            pl.BlockSpec((D2, D1), lambda i: (0, 0)),
            pl.BlockSpec((D2, D2), lambda i: (0, 0)),
        ],
        out_specs=[
            pl.BlockSpec((EB, D1), lambda i: (i, 0)),
            pl.BlockSpec((EB, D2), lambda i: (i, 0)),
        ],
        out_shape=[
            jax.ShapeDtypeStruct((E, D1), jnp.float32),
            jax.ShapeDtypeStruct((E, D2), jnp.float32),
        ],
    )(edge_attr, We1, We2)

    acc1 = _sc_edge1(xl1, src, dst, ef1, att1)

    xl2 = pl.pallas_call(
        _norm_lin2_body,
        grid=(N // RB,),
        in_specs=[
            pl.BlockSpec((NC, RB, D1P), lambda i: (0, i, 0)),
            pl.BlockSpec((D1,), lambda i: (0,)),
            pl.BlockSpec((D1, D2), lambda i: (0, 0)),
            pl.BlockSpec((D2,), lambda i: (0,)),
        ],
        out_specs=pl.BlockSpec((RB, D2P), lambda i: (i, 0)),
        out_shape=jax.ShapeDtypeStruct((N, D2P), jnp.float32),
    )(acc1, bias1, W2, b2)

    acc2 = _sc_edge2(xl2, src, dst, ef2, att2)

    action_logits = pl.pallas_call(
        _norm2_body,
        grid=(N // RB,),
        in_specs=[
            pl.BlockSpec((NC, RB, D2P), lambda i: (0, i, 0)),
            pl.BlockSpec((D2,), lambda i: (0,)),
        ],
        out_specs=pl.BlockSpec((RB, D2), lambda i: (i, 0)),
        out_shape=jax.ShapeDtypeStruct((N, D2), jnp.float32),
    )(acc2, bias2)

    flat = action_logits.reshape(-1)
    skey = jax.random.key(42)
    idx = jax.random.categorical(skey, flat)
    log_prob = jax.nn.log_softmax(flat)[idx]
    sel_node, sel_action = jnp.unravel_index(idx, action_logits.shape)
    return (sel_node, sel_action, log_prob)


# speedup vs baseline: 11.1013x; 1.1837x over previous
import functools

import jax
import jax.numpy as jnp
from jax import lax
from jax.experimental import pallas as pl
from jax.experimental.pallas import tpu as pltpu
from jax.experimental.pallas import tpu_sc as plsc

N = 10000
E = 320000
D1 = 128
D2 = 16
D1P = D1 + 16
D2P = D2 + 16
NC, NS, L = 2, 16, 16
NW = NC * NS


def _lin1_body(x_ref, w_ref, b_ref, o_ref):
    rb = x_ref.shape[0]
    feat = (
        jnp.dot(x_ref[...], w_ref[...], preferred_element_type=jnp.float32)
        + b_ref[...]
    )
    col = lax.broadcasted_iota(jnp.int32, (rb, D1P - D1), 1)
    pad = jnp.where(col == 0, 1.0, 0.0)
    o_ref[...] = jnp.concatenate([feat, pad], axis=1)


def _edge_proj_body(ea_ref, we1_ref, we2_ref, o1_ref, o2_ref):
    ea = ea_ref[...]
    o1_ref[...] = jnp.dot(ea, we1_ref[...], preferred_element_type=jnp.float32)
    o2_ref[...] = jnp.dot(ea, we2_ref[...], preferred_element_type=jnp.float32)


def _norm_lin2_body(acc_ref, bias1_ref, w2_ref, b2_ref, o_ref):
    rb = o_ref.shape[0]
    acc = acc_ref[0] + acc_ref[1]
    den = acc[:, D1] + 1e-16
    lat = acc[:, :D1] / den[:, None] + bias1_ref[...]
    feat = (
        jnp.dot(lat, w2_ref[...], preferred_element_type=jnp.float32)
        + b2_ref[...]
    )
    col = lax.broadcasted_iota(jnp.int32, (rb, D2P - D2), 1)
    pad = jnp.where(col == 0, 1.0, 0.0)
    o_ref[...] = jnp.concatenate([feat, pad], axis=1)


def _norm2_body(acc_ref, bias2_ref, o_ref):
    acc = acc_ref[0] + acc_ref[1]
    den = acc[:, D2] + 1e-16
    o_ref[...] = acc[:, :D2] / den[:, None] + bias2_ref[...]


_MESH = plsc.VectorSubcoreMesh(core_axis_name="c", subcore_axis_name="s")


def _sc_edge_body(dp, dfull, ch, xl_hbm, src_hbm, dst_hbm, ef_hbm, att_hbm,
                  acc_out,
                  acc_sh, srcv, dstv, sbuf, ab, bb, efb, attv,
                  sem_i, sem_a, sem_b, sem_e, sem_s):
    cid = lax.axis_index("c")
    sid = lax.axis_index("s")
    wid = sid * NC + cid
    nj = dfull // L
    njp = dp // L
    nchunks = E // ch
    total = nchunks // NW
    assert total * NW == nchunks

    zero16 = jnp.zeros((L,), jnp.float32)

    def _zrow(i, carry):
        for j in range(njp):
            ab[0][i, pl.ds(L * j, L)] = zero16
        return carry

    lax.fori_loop(0, ch, _zrow, 0)

    base_row = sid * 624
    _sizes, _offs, _rem = [], [], 624
    while _rem:
        _offs.append(624 - _rem)
        _sizes.append(min(_rem, ch))
        _rem -= _sizes[-1]
    for sz, off in zip(_sizes, _offs):
        pltpu.sync_copy(ab[0].at[pl.ds(0, sz)],
                        acc_sh.at[pl.ds(base_row + off, sz)])

    @pl.when(sid == NS - 1)
    def _():
        pltpu.sync_copy(ab[0].at[pl.ds(0, 16)], acc_sh.at[pl.ds(624 * NS, 16)])

    plsc.subcore_barrier()

    pltpu.sync_copy(att_hbm, attv)
    att_regs = [attv[pl.ds(L * j, L)] for j in range(nj)]
    lane = lax.iota(jnp.int32, L)

    def _ebase(i):
        return (wid + NW * i) * ch

    def _idx_start(p, i):
        eb = _ebase(i)
        pltpu.async_copy(src_hbm.at[pl.ds(eb, ch)], srcv[p], sem_i[p])
        pltpu.async_copy(dst_hbm.at[pl.ds(eb, ch)], dstv[p], sem_i[p])

    def _idx_wait(p):
        pltpu.make_async_copy(src_hbm.at[pl.ds(0, ch)], srcv[p],
                              sem_i[p]).wait()
        pltpu.make_async_copy(dst_hbm.at[pl.ds(0, ch)], dstv[p],
                              sem_i[p]).wait()

    def _gather_start(p, i):
        eb = _ebase(i)
        pltpu.async_copy(xl_hbm.at[srcv[p]], ab[p], sem_a[p])
        pltpu.async_copy(xl_hbm.at[dstv[p]], bb[p], sem_b[p])
        pltpu.async_copy(ef_hbm.at[pl.ds(eb, ch)], efb[p], sem_e[p])

    def _gather_wait(p):
        pltpu.make_async_copy(xl_hbm.at[srcv[p]], ab[p], sem_a[p]).wait()
        pltpu.make_async_copy(xl_hbm.at[dstv[p]], bb[p], sem_b[p]).wait()
        pltpu.make_async_copy(ef_hbm.at[pl.ds(0, ch)], efb[p],
                              sem_e[p]).wait()

    def _scatter_wait(p):
        pltpu.make_async_copy(ab[p], acc_sh.at[sbuf[p]], sem_s[p]).wait()

    def _compute(p):
        for off in range(0, ch - L + 1, L):
            sbuf[p][pl.ds(off, L)] = dstv[p][pl.ds(off, L)]
        if ch % L:
            off = ch - L
            sbuf[p][pl.ds(off, L)] = dstv[p][pl.ds(off, L)]

        def _edge(e, cc):
            a_regs = []
            s = zero16
            for j in range(nj):
                sl = pl.ds(L * j, L)
                a = ab[p][e, sl]
                a_regs.append(a)
                v = a + bb[p][e, sl] + efb[p][e, sl]
                lv = jnp.where(v > 0, v, 0.2 * v)
                s = s + lv * att_regs[j]
            wv = jnp.exp(jnp.broadcast_to(jnp.sum(s), (L,)))
            for j in range(nj):
                sl = pl.ds(L * j, L)
                ab[p][e, sl] = a_regs[j] * wv
            ab[p][e, pl.ds(nj * L, L)] = jnp.where(lane == 0, wv, 0.0)
            return cc

        lax.fori_loop(0, ch, _edge, 0)

    def _scatter_start(p):
        pltpu.async_copy(ab[p], acc_sh.at[sbuf[p]], sem_s[p], add=True)

    _idx_start(0, 0)
    _idx_start(1, 1)
    _idx_wait(0)
    _gather_start(0, 0)

    def _iter(i, p):
        @pl.when(i >= 1)
        def _():
            _scatter_wait(1 - p)

        @pl.when(i + 1 < total)
        def _():
            _idx_wait(1 - p)
            _gather_start(1 - p, i + 1)

        _gather_wait(p)

        @pl.when(i + 2 < total)
        def _():
            _idx_start(p, i + 2)

        _compute(p)
        _scatter_start(p)

    def _pair(t, carry):
        _iter(2 * t, 0)
        _iter(2 * t + 1, 1)
        return carry

    lax.fori_loop(0, total // 2, _pair, 0)
    if total % 2:
        _iter(total - 1, 0)
        _scatter_wait(0)
    else:
        _scatter_wait(1)

    plsc.subcore_barrier()

    for sz, off in zip(_sizes, _offs):
        sl = pl.ds(base_row + off, sz)
        pltpu.sync_copy(acc_sh.at[sl], acc_out.at[cid, sl])

    @pl.when(sid == NS - 1)
    def _():
        sl = pl.ds(624 * NS, 16)
        pltpu.sync_copy(acc_sh.at[sl], acc_out.at[cid, sl])


def _make_sc_edge(dp, dfull, ch):
    return functools.partial(
        pl.kernel,
        out_type=jax.ShapeDtypeStruct((NC, N, dp), jnp.float32),
        mesh=_MESH,
        scratch_types=[
            pltpu.VMEM_SHARED((N, dp), jnp.float32),
            [pltpu.VMEM((ch,), jnp.int32)] * 2,
            [pltpu.VMEM((ch,), jnp.int32)] * 2,
            [pltpu.VMEM((ch,), jnp.int32)] * 2,
            [pltpu.VMEM((ch, dp), jnp.float32)] * 2,
            [pltpu.VMEM((ch, dp), jnp.float32)] * 2,
            [pltpu.VMEM((ch, dfull), jnp.float32)] * 2,
            pltpu.VMEM((dfull,), jnp.float32),
            [pltpu.SemaphoreType.DMA] * 2,
            [pltpu.SemaphoreType.DMA] * 2,
            [pltpu.SemaphoreType.DMA] * 2,
            [pltpu.SemaphoreType.DMA] * 2,
            [pltpu.SemaphoreType.DMA] * 2,
        ],
        compiler_params=pltpu.CompilerParams(use_tc_tiling_on_sc=False,
                                             needs_layout_passes=False),
    )(functools.partial(_sc_edge_body, dp, dfull, ch))


_sc_edge1 = _make_sc_edge(D1P, D1, 40)
_sc_edge2 = _make_sc_edge(D2P, D2, 80)


def kernel(x, edge_index, edge_attr, W1, b1, We1, att1, bias1,
           W2, b2, We2, att2, bias2):
    src = edge_index[0]
    dst = edge_index[1]

    RB = 400
    xl1 = pl.pallas_call(
        _lin1_body,
        grid=(N // RB,),
        in_specs=[
            pl.BlockSpec((RB, D1), lambda i: (i, 0)),
            pl.BlockSpec((D1, D1), lambda i: (0, 0)),
            pl.BlockSpec((D1,), lambda i: (0,)),
        ],
        out_specs=pl.BlockSpec((RB, D1P), lambda i: (i, 0)),
        out_shape=jax.ShapeDtypeStruct((N, D1P), jnp.float32),
    )(x, W1, b1)

    EB = 2000
    ef1, ef2 = pl.pallas_call(
        _edge_proj_body,
        grid=(E // EB,),
        in_specs=[
            pl.BlockSpec((EB, D2), lambda i: (i, 0)),
            pl.BlockSpec((D2, D1), lambda i: (0, 0)),
            pl.BlockSpec((D2, D2), lambda i: (0, 0)),
        ],
        out_specs=[
            pl.BlockSpec((EB, D1), lambda i: (i, 0)),
            pl.BlockSpec((EB, D2), lambda i: (i, 0)),
        ],
        out_shape=[
            jax.ShapeDtypeStruct((E, D1), jnp.float32),
            jax.ShapeDtypeStruct((E, D2), jnp.float32),
        ],
    )(edge_attr, We1, We2)

    acc1 = _sc_edge1(xl1, src, dst, ef1, att1)

    xl2 = pl.pallas_call(
        _norm_lin2_body,
        grid=(N // RB,),
        in_specs=[
            pl.BlockSpec((NC, RB, D1P), lambda i: (0, i, 0)),
            pl.BlockSpec((D1,), lambda i: (0,)),
            pl.BlockSpec((D1, D2), lambda i: (0, 0)),
            pl.BlockSpec((D2,), lambda i: (0,)),
        ],
        out_specs=pl.BlockSpec((RB, D2P), lambda i: (i, 0)),
        out_shape=jax.ShapeDtypeStruct((N, D2P), jnp.float32),
    )(acc1, bias1, W2, b2)

    acc2 = _sc_edge2(xl2, src, dst, ef2, att2)

    action_logits = pl.pallas_call(
        _norm2_body,
        grid=(N // RB,),
        in_specs=[
            pl.BlockSpec((NC, RB, D2P), lambda i: (0, i, 0)),
            pl.BlockSpec((D2,), lambda i: (0,)),
        ],
        out_specs=pl.BlockSpec((RB, D2), lambda i: (i, 0)),
        out_shape=jax.ShapeDtypeStruct((N, D2), jnp.float32),
    )(acc2, bias2)

    flat = action_logits.reshape(-1)
    skey = jax.random.key(42)
    idx = jax.random.categorical(skey, flat)
    log_prob = jax.nn.log_softmax(flat)[idx]
    sel_node, sel_action = jnp.unravel_index(idx, action_logits.shape)
    return (sel_node, sel_action, log_prob)


# speedup vs baseline: 11.7760x; 1.0608x over previous
import functools

import jax
import jax.numpy as jnp
from jax import lax
from jax.experimental import pallas as pl
from jax.experimental.pallas import tpu as pltpu
from jax.experimental.pallas import tpu_sc as plsc

N = 10000
E = 320000
D1 = 128
D2 = 16
D1P = D1 + 16
D2P = D2 + 16
NC, NS, L = 2, 16, 16
NW = NC * NS


def _lin1_body(x_ref, w_ref, b_ref, o_ref):
    rb = x_ref.shape[0]
    feat = (
        jnp.dot(x_ref[...], w_ref[...], preferred_element_type=jnp.float32)
        + b_ref[...]
    )
    col = lax.broadcasted_iota(jnp.int32, (rb, D1P - D1), 1)
    pad = jnp.where(col == 0, 1.0, 0.0)
    o_ref[...] = jnp.concatenate([feat, pad], axis=1)


def _edge_proj_body(ea_ref, we1_ref, we2_ref, o1_ref, o2_ref):
    ea = ea_ref[...]
    o1_ref[...] = jnp.dot(ea, we1_ref[...], preferred_element_type=jnp.float32)
    o2_ref[...] = jnp.dot(ea, we2_ref[...], preferred_element_type=jnp.float32)


def _norm_lin2_body(acc_ref, bias1_ref, w2_ref, b2_ref, o_ref):
    rb = o_ref.shape[0]
    acc = acc_ref[0] + acc_ref[1]
    den = acc[:, D1] + 1e-16
    lat = acc[:, :D1] / den[:, None] + bias1_ref[...]
    feat = (
        jnp.dot(lat, w2_ref[...], preferred_element_type=jnp.float32)
        + b2_ref[...]
    )
    col = lax.broadcasted_iota(jnp.int32, (rb, D2P - D2), 1)
    pad = jnp.where(col == 0, 1.0, 0.0)
    o_ref[...] = jnp.concatenate([feat, pad], axis=1)


def _norm2_body(acc_ref, bias2_ref, o_ref):
    acc = acc_ref[0] + acc_ref[1]
    den = acc[:, D2] + 1e-16
    o_ref[...] = acc[:, :D2] / den[:, None] + bias2_ref[...]


_MESH = plsc.VectorSubcoreMesh(core_axis_name="c", subcore_axis_name="s")


def _sc_edge_body(dp, dfull, ch, xl_hbm, src_hbm, dst_hbm, ef_hbm, att_hbm,
                  acc_out,
                  acc_sh, srcv, dstv, sbuf, ab, bb, efb, ob, attv,
                  sem_i, sem_a, sem_b, sem_e, sem_s):
    cid = lax.axis_index("c")
    sid = lax.axis_index("s")
    wid = sid * NC + cid
    nj = dfull // L
    njp = dp // L
    nchunks = E // ch
    total = nchunks // NW
    assert total * NW == nchunks

    zero16 = jnp.zeros((L,), jnp.float32)

    def _zrow(i, carry):
        for j in range(njp):
            ob[i, pl.ds(L * j, L)] = zero16
        return carry

    lax.fori_loop(0, ch, _zrow, 0)

    base_row = sid * 624
    _sizes, _offs, _rem = [], [], 624
    while _rem:
        _offs.append(624 - _rem)
        _sizes.append(min(_rem, ch))
        _rem -= _sizes[-1]
    for sz, off in zip(_sizes, _offs):
        pltpu.sync_copy(ob.at[pl.ds(0, sz)],
                        acc_sh.at[pl.ds(base_row + off, sz)])

    @pl.when(sid == NS - 1)
    def _():
        pltpu.sync_copy(ob.at[pl.ds(0, 16)], acc_sh.at[pl.ds(624 * NS, 16)])

    plsc.subcore_barrier()

    pltpu.sync_copy(att_hbm, attv)
    att_regs = [attv[pl.ds(L * j, L)] for j in range(nj)]
    lane = lax.iota(jnp.int32, L)

    def _ebase(i):
        return (wid + NW * i) * ch

    def _idx_start(p, i):
        eb = _ebase(i)
        pltpu.async_copy(src_hbm.at[pl.ds(eb, ch)], srcv[p], sem_i[p])
        pltpu.async_copy(dst_hbm.at[pl.ds(eb, ch)], dstv[p], sem_i[p])

    def _idx_wait(p):
        pltpu.make_async_copy(src_hbm.at[pl.ds(0, ch)], srcv[p],
                              sem_i[p]).wait()
        pltpu.make_async_copy(dst_hbm.at[pl.ds(0, ch)], dstv[p],
                              sem_i[p]).wait()

    def _gather_start(p, i):
        eb = _ebase(i)
        pltpu.async_copy(xl_hbm.at[srcv[p]], ab[p], sem_a[p])
        pltpu.async_copy(xl_hbm.at[dstv[p]], bb[p], sem_b[p])
        pltpu.async_copy(ef_hbm.at[pl.ds(eb * dfull, ch * dfull)], efb[p],
                         sem_e[p])

    def _gather_wait(p):
        pltpu.make_async_copy(xl_hbm.at[srcv[p]], ab[p], sem_a[p]).wait()
        pltpu.make_async_copy(xl_hbm.at[dstv[p]], bb[p], sem_b[p]).wait()
        pltpu.make_async_copy(ef_hbm.at[pl.ds(0, ch * dfull)], efb[p],
                              sem_e[p]).wait()

    def _scatter_wait():
        pltpu.make_async_copy(ob, acc_sh.at[sbuf], sem_s).wait()

    def _compute(p):
        for off in range(0, ch - L + 1, L):
            sbuf[pl.ds(off, L)] = dstv[p][pl.ds(off, L)]
        if ch % L:
            off = ch - L
            sbuf[pl.ds(off, L)] = dstv[p][pl.ds(off, L)]

        def _edge(e, cc):
            s = zero16
            a_regs = []
            for j in range(nj):
                sl = pl.ds(L * j, L)
                a = ab[p][e, sl]
                a_regs.append(a)
                v = a + bb[p][e, sl] + efb[p][pl.ds(e * dfull + L * j, L)]
                lv = jnp.where(v > 0, v, 0.2 * v)
                s = s + lv * att_regs[j]
            wv = jnp.exp(jnp.broadcast_to(jnp.sum(s), (L,)))
            for j in range(nj):
                ob[e, pl.ds(L * j, L)] = a_regs[j] * wv
            ob[e, pl.ds(nj * L, L)] = jnp.where(lane == 0, wv, 0.0)
            return cc

        lax.fori_loop(0, ch, _edge, 0)

    def _scatter_start():
        pltpu.async_copy(ob, acc_sh.at[sbuf], sem_s, add=True)

    _idx_start(0, 0)
    _idx_start(1, 1)
    _idx_wait(0)
    _gather_start(0, 0)

    def _iter(i, p):
        @pl.when(i + 1 < total)
        def _():
            _idx_wait(1 - p)
            _gather_start(1 - p, i + 1)

        _gather_wait(p)

        @pl.when(i + 2 < total)
        def _():
            _idx_start(p, i + 2)

        @pl.when(i >= 1)
        def _():
            _scatter_wait()

        _compute(p)
        _scatter_start()

    def _pair(t, carry):
        _iter(2 * t, 0)
        _iter(2 * t + 1, 1)
        return carry

    lax.fori_loop(0, total // 2, _pair, 0)
    if total % 2:
        _iter(total - 1, 0)
    _scatter_wait()

    plsc.subcore_barrier()

    for sz, off in zip(_sizes, _offs):
        sl = pl.ds(base_row + off, sz)
        pltpu.sync_copy(acc_sh.at[sl], acc_out.at[cid, sl])

    @pl.when(sid == NS - 1)
    def _():
        sl = pl.ds(624 * NS, 16)
        pltpu.sync_copy(acc_sh.at[sl], acc_out.at[cid, sl])


def _make_sc_edge(dp, dfull, ch):
    return functools.partial(
        pl.kernel,
        out_type=jax.ShapeDtypeStruct((NC, N, dp), jnp.float32),
        mesh=_MESH,
        scratch_types=[
            pltpu.VMEM_SHARED((N, dp), jnp.float32),
            [pltpu.VMEM((ch,), jnp.int32)] * 2,
            [pltpu.VMEM((ch,), jnp.int32)] * 2,
            pltpu.VMEM((ch,), jnp.int32),
            [pltpu.VMEM((ch, dp), jnp.float32)] * 2,
            [pltpu.VMEM((ch, dp), jnp.float32)] * 2,
            [pltpu.VMEM((ch * dfull,), jnp.float32)] * 2,
            pltpu.VMEM((ch, dp), jnp.float32),
            pltpu.VMEM((dfull,), jnp.float32),
            [pltpu.SemaphoreType.DMA] * 2,
            [pltpu.SemaphoreType.DMA] * 2,
            [pltpu.SemaphoreType.DMA] * 2,
            [pltpu.SemaphoreType.DMA] * 2,
            pltpu.SemaphoreType.DMA,
        ],
        compiler_params=pltpu.CompilerParams(use_tc_tiling_on_sc=False,
                                             needs_layout_passes=False),
    )(functools.partial(_sc_edge_body, dp, dfull, ch))


_sc_edge1 = _make_sc_edge(D1P, D1, 40)
_sc_edge2 = _make_sc_edge(D2P, D2, 80)


def kernel(x, edge_index, edge_attr, W1, b1, We1, att1, bias1,
           W2, b2, We2, att2, bias2):
    src = edge_index[0]
    dst = edge_index[1]

    RB = 400
    xl1 = pl.pallas_call(
        _lin1_body,
        grid=(N // RB,),
        in_specs=[
            pl.BlockSpec((RB, D1), lambda i: (i, 0)),
            pl.BlockSpec((D1, D1), lambda i: (0, 0)),
            pl.BlockSpec((D1,), lambda i: (0,)),
        ],
        out_specs=pl.BlockSpec((RB, D1P), lambda i: (i, 0)),
        out_shape=jax.ShapeDtypeStruct((N, D1P), jnp.float32),
    )(x, W1, b1)

    EB = 5000
    ef1, ef2 = pl.pallas_call(
        _edge_proj_body,
        grid=(E // EB,),
        in_specs=[
            pl.BlockSpec((EB, D2), lambda i: (i, 0)),
            pl.BlockSpec((D2, D1), lambda i: (0, 0)),
            pl.BlockSpec((D2, D2), lambda i: (0, 0)),
        ],
        out_specs=[
            pl.BlockSpec((EB, D1), lambda i: (i, 0)),
            pl.BlockSpec((EB, D2), lambda i: (i, 0)),
        ],
        out_shape=[
            jax.ShapeDtypeStruct((E, D1), jnp.float32),
            jax.ShapeDtypeStruct((E, D2), jnp.float32),
        ],
    )(edge_attr, We1, We2)

    acc1 = _sc_edge1(xl1, src, dst, ef1.reshape(-1), att1)

    xl2 = pl.pallas_call(
        _norm_lin2_body,
        grid=(N // RB,),
        in_specs=[
            pl.BlockSpec((NC, RB, D1P), lambda i: (0, i, 0)),
            pl.BlockSpec((D1,), lambda i: (0,)),
            pl.BlockSpec((D1, D2), lambda i: (0, 0)),
            pl.BlockSpec((D2,), lambda i: (0,)),
        ],
        out_specs=pl.BlockSpec((RB, D2P), lambda i: (i, 0)),
        out_shape=jax.ShapeDtypeStruct((N, D2P), jnp.float32),
    )(acc1, bias1, W2, b2)

    acc2 = _sc_edge2(xl2, src, dst, ef2.reshape(-1), att2)

    action_logits = pl.pallas_call(
        _norm2_body,
        grid=(N // RB,),
        in_specs=[
            pl.BlockSpec((NC, RB, D2P), lambda i: (0, i, 0)),
            pl.BlockSpec((D2,), lambda i: (0,)),
        ],
        out_specs=pl.BlockSpec((RB, D2), lambda i: (i, 0)),
        out_shape=jax.ShapeDtypeStruct((N, D2), jnp.float32),
    )(acc2, bias2)

    flat = action_logits.reshape(-1)
    skey = jax.random.key(42)
    idx = jax.random.categorical(skey, flat)
    log_prob = jax.nn.log_softmax(flat)[idx]
    sel_node, sel_action = jnp.unravel_index(idx, action_logits.shape)
    return (sel_node, sel_action, log_prob)


# speedup vs baseline: 12.3692x; 1.0504x over previous
import functools

import jax
import jax.numpy as jnp
from jax import lax
from jax.experimental import pallas as pl
from jax.experimental.pallas import tpu as pltpu
from jax.experimental.pallas import tpu_sc as plsc

N = 10000
E = 320000
D1 = 128
D2 = 16
D1P = D1 + 16
D2P = D2 + 16
NC, NS, L = 2, 16, 16
NW = NC * NS


def _lin1_body(x_ref, w_ref, b_ref, o_ref):
    rb = x_ref.shape[0]
    feat = (
        jnp.dot(x_ref[...], w_ref[...], preferred_element_type=jnp.float32)
        + b_ref[...]
    )
    col = lax.broadcasted_iota(jnp.int32, (rb, D1P - D1), 1)
    pad = jnp.where(col == 0, 1.0, 0.0)
    o_ref[...] = jnp.concatenate([feat, pad], axis=1)


def _edge_proj_body(ea_ref, we1_ref, we2_ref, o1_ref, o2_ref):
    dn = (((0,), (0,)), ((), ()))
    ea_t = ea_ref[...]
    o1_ref[...] = lax.dot_general(ea_t, we1_ref[...], dn,
                                  preferred_element_type=jnp.float32)
    o2_ref[...] = lax.dot_general(ea_t, we2_ref[...], dn,
                                  preferred_element_type=jnp.float32)


def _norm_lin2_body(acc_ref, bias1_ref, w2_ref, b2_ref, o_ref):
    rb = o_ref.shape[0]
    acc = acc_ref[0] + acc_ref[1]
    den = acc[:, D1] + 1e-16
    lat = acc[:, :D1] / den[:, None] + bias1_ref[...]
    feat = (
        jnp.dot(lat, w2_ref[...], preferred_element_type=jnp.float32)
        + b2_ref[...]
    )
    col = lax.broadcasted_iota(jnp.int32, (rb, D2P - D2), 1)
    pad = jnp.where(col == 0, 1.0, 0.0)
    o_ref[...] = jnp.concatenate([feat, pad], axis=1)


def _norm2_body(acc_ref, bias2_ref, o_ref):
    acc = acc_ref[0] + acc_ref[1]
    den = acc[:, D2] + 1e-16
    o_ref[...] = acc[:, :D2] / den[:, None] + bias2_ref[...]


_MESH = plsc.VectorSubcoreMesh(core_axis_name="c", subcore_axis_name="s")


def _sc_edge_body(dp, dfull, ch, xl_hbm, src_hbm, dst_hbm, ef_hbm, att_hbm,
                  acc_out,
                  acc_sh, srcv, dstv, sbuf, ab, bb, efb, ob, attv,
                  sem_i, sem_a, sem_b, sem_e, sem_s):
    cid = lax.axis_index("c")
    sid = lax.axis_index("s")
    wid = sid * NC + cid
    nj = dfull // L
    njp = dp // L
    nchunks = E // ch
    total = nchunks // NW
    assert total * NW == nchunks

    zero16 = jnp.zeros((L,), jnp.float32)

    def _zrow(i, carry):
        for j in range(njp):
            ob[i, pl.ds(L * j, L)] = zero16
        return carry

    lax.fori_loop(0, ch, _zrow, 0)

    base_row = sid * 624
    _sizes, _offs, _rem = [], [], 624
    while _rem:
        _offs.append(624 - _rem)
        _sizes.append(min(_rem, ch))
        _rem -= _sizes[-1]
    for sz, off in zip(_sizes, _offs):
        pltpu.sync_copy(ob.at[pl.ds(0, sz)],
                        acc_sh.at[pl.ds(base_row + off, sz)])

    @pl.when(sid == NS - 1)
    def _():
        pltpu.sync_copy(ob.at[pl.ds(0, 16)], acc_sh.at[pl.ds(624 * NS, 16)])

    plsc.subcore_barrier()

    pltpu.sync_copy(att_hbm, attv)
    att_regs = [attv[pl.ds(L * j, L)] for j in range(nj)]
    lane = lax.iota(jnp.int32, L)

    def _ebase(i):
        return (wid + NW * i) * ch

    def _idx_start(p, i):
        eb = _ebase(i)
        pltpu.async_copy(src_hbm.at[pl.ds(eb, ch)], srcv[p], sem_i[p])
        pltpu.async_copy(dst_hbm.at[pl.ds(eb, ch)], dstv[p], sem_i[p])

    def _idx_wait(p):
        pltpu.make_async_copy(src_hbm.at[pl.ds(0, ch)], srcv[p],
                              sem_i[p]).wait()
        pltpu.make_async_copy(dst_hbm.at[pl.ds(0, ch)], dstv[p],
                              sem_i[p]).wait()

    def _gather_start(p, i):
        eb = _ebase(i)
        pltpu.async_copy(xl_hbm.at[srcv[p]], ab[p], sem_a[p])
        pltpu.async_copy(xl_hbm.at[dstv[p]], bb[p], sem_b[p])
        pltpu.async_copy(ef_hbm.at[pl.ds(eb, ch), pl.ds(0, dfull)], efb[p],
                         sem_e[p])

    def _gather_wait(p):
        pltpu.make_async_copy(xl_hbm.at[srcv[p]], ab[p], sem_a[p]).wait()
        pltpu.make_async_copy(xl_hbm.at[dstv[p]], bb[p], sem_b[p]).wait()
        pltpu.make_async_copy(ef_hbm.at[pl.ds(0, ch), pl.ds(0, dfull)],
                              efb[p], sem_e[p]).wait()

    def _scatter_wait():
        pltpu.make_async_copy(ob, acc_sh.at[sbuf], sem_s).wait()

    def _compute(p):
        for off in range(0, ch - L + 1, L):
            sbuf[pl.ds(off, L)] = dstv[p][pl.ds(off, L)]
        if ch % L:
            off = ch - L
            sbuf[pl.ds(off, L)] = dstv[p][pl.ds(off, L)]

        def _edge(e, cc):
            s = zero16
            a_regs = []
            for j in range(nj):
                sl = pl.ds(L * j, L)
                a = ab[p][e, sl]
                a_regs.append(a)
                v = a + bb[p][e, sl] + efb[p][e, sl]
                lv = jnp.where(v > 0, v, 0.2 * v)
                s = s + lv * att_regs[j]
            wv = jnp.exp(jnp.broadcast_to(jnp.sum(s), (L,)))
            for j in range(nj):
                ob[e, pl.ds(L * j, L)] = a_regs[j] * wv
            ob[e, pl.ds(nj * L, L)] = jnp.where(lane == 0, wv, 0.0)
            return cc

        lax.fori_loop(0, ch, _edge, 0, unroll=2 if dfull > L else 8)

    def _scatter_start():
        pltpu.async_copy(ob, acc_sh.at[sbuf], sem_s, add=True)

    _idx_start(0, 0)
    _idx_start(1, 1)
    _idx_wait(0)
    _gather_start(0, 0)

    def _iter(i, p):
        @pl.when(i + 1 < total)
        def _():
            _idx_wait(1 - p)
            _gather_start(1 - p, i + 1)

        _gather_wait(p)

        @pl.when(i + 2 < total)
        def _():
            _idx_start(p, i + 2)

        @pl.when(i >= 1)
        def _():
            _scatter_wait()

        _compute(p)
        _scatter_start()

    def _pair(t, carry):
        _iter(2 * t, 0)
        _iter(2 * t + 1, 1)
        return carry

    lax.fori_loop(0, total // 2, _pair, 0)
    if total % 2:
        _iter(total - 1, 0)
    _scatter_wait()

    plsc.subcore_barrier()

    for sz, off in zip(_sizes, _offs):
        sl = pl.ds(base_row + off, sz)
        pltpu.sync_copy(acc_sh.at[sl], acc_out.at[cid, sl])

    @pl.when(sid == NS - 1)
    def _():
        sl = pl.ds(624 * NS, 16)
        pltpu.sync_copy(acc_sh.at[sl], acc_out.at[cid, sl])


def _make_sc_edge(dp, dfull, ch):
    return functools.partial(
        pl.kernel,
        out_type=jax.ShapeDtypeStruct((NC, N, dp), jnp.float32),
        mesh=_MESH,
        scratch_types=[
            pltpu.VMEM_SHARED((N, dp), jnp.float32),
            [pltpu.VMEM((ch,), jnp.int32)] * 2,
            [pltpu.VMEM((ch,), jnp.int32)] * 2,
            pltpu.VMEM((ch,), jnp.int32),
            [pltpu.VMEM((ch, dp), jnp.float32)] * 2,
            [pltpu.VMEM((ch, dp), jnp.float32)] * 2,
            [pltpu.VMEM((ch, dfull), jnp.float32)] * 2,
            pltpu.VMEM((ch, dp), jnp.float32),
            pltpu.VMEM((dfull,), jnp.float32),
            [pltpu.SemaphoreType.DMA] * 2,
            [pltpu.SemaphoreType.DMA] * 2,
            [pltpu.SemaphoreType.DMA] * 2,
            [pltpu.SemaphoreType.DMA] * 2,
            pltpu.SemaphoreType.DMA,
        ],
        compiler_params=pltpu.CompilerParams(use_tc_tiling_on_sc=False,
                                             needs_layout_passes=False),
    )(functools.partial(_sc_edge_body, dp, dfull, ch))


_sc_edge1 = _make_sc_edge(D1P, D1, 40)
_sc_edge2 = _make_sc_edge(D2P, D2, 80)


def kernel(x, edge_index, edge_attr, W1, b1, We1, att1, bias1,
           W2, b2, We2, att2, bias2):
    src = edge_index[0]
    dst = edge_index[1]

    RB = 400
    xl1 = pl.pallas_call(
        _lin1_body,
        grid=(N // RB,),
        in_specs=[
            pl.BlockSpec((RB, D1), lambda i: (i, 0)),
            pl.BlockSpec((D1, D1), lambda i: (0, 0)),
            pl.BlockSpec((D1,), lambda i: (0,)),
        ],
        out_specs=pl.BlockSpec((RB, D1P), lambda i: (i, 0)),
        out_shape=jax.ShapeDtypeStruct((N, D1P), jnp.float32),
    )(x, W1, b1)

    EB = 6400
    We2p = jnp.pad(We2, ((0, 0), (0, D1 - D2)))
    ef1, ef2 = pl.pallas_call(
        _edge_proj_body,
        grid=(E // EB,),
        in_specs=[
            pl.BlockSpec((D2, EB), lambda i: (0, i)),
            pl.BlockSpec((D2, D1), lambda i: (0, 0)),
            pl.BlockSpec((D2, D1), lambda i: (0, 0)),
        ],
        out_specs=[
            pl.BlockSpec((EB, D1), lambda i: (i, 0)),
            pl.BlockSpec((EB, D1), lambda i: (i, 0)),
        ],
        out_shape=[
            jax.ShapeDtypeStruct((E, D1), jnp.float32),
            jax.ShapeDtypeStruct((E, D1), jnp.float32),
        ],
    )(edge_attr.T, We1, We2p)

    acc1 = _sc_edge1(xl1, src, dst, ef1, att1)

    xl2 = pl.pallas_call(
        _norm_lin2_body,
        grid=(N // RB,),
        in_specs=[
            pl.BlockSpec((NC, RB, D1P), lambda i: (0, i, 0)),
            pl.BlockSpec((D1,), lambda i: (0,)),
            pl.BlockSpec((D1, D2), lambda i: (0, 0)),
            pl.BlockSpec((D2,), lambda i: (0,)),
        ],
        out_specs=pl.BlockSpec((RB, D2P), lambda i: (i, 0)),
        out_shape=jax.ShapeDtypeStruct((N, D2P), jnp.float32),
    )(acc1, bias1, W2, b2)

    acc2 = _sc_edge2(xl2, src, dst, ef2, att2)

    action_logits = pl.pallas_call(
        _norm2_body,
        grid=(N // RB,),
        in_specs=[
            pl.BlockSpec((NC, RB, D2P), lambda i: (0, i, 0)),
            pl.BlockSpec((D2,), lambda i: (0,)),
        ],
        out_specs=pl.BlockSpec((RB, D2), lambda i: (i, 0)),
        out_shape=jax.ShapeDtypeStruct((N, D2), jnp.float32),
    )(acc2, bias2)

    flat = action_logits.reshape(-1)
    skey = jax.random.key(42)
    idx = jax.random.categorical(skey, flat)
    log_prob = jax.nn.log_softmax(flat)[idx]
    sel_node, sel_action = jnp.unravel_index(idx, action_logits.shape)
    return (sel_node, sel_action, log_prob)


# speedup vs baseline: 13.4438x; 1.0869x over previous
import functools

import jax
import jax.numpy as jnp
from jax import lax
from jax.experimental import pallas as pl
from jax.experimental.pallas import tpu as pltpu
from jax.experimental.pallas import tpu_sc as plsc

N = 10000
E = 320000
D1 = 128
D2 = 16
D1P = D1 + 16
D2P = D2 + 16
NC, NS, L = 2, 16, 16
NW = NC * NS


def _lin1_body(x_ref, w_ref, b_ref, o_ref):
    rb = x_ref.shape[0]
    feat = (
        jnp.dot(x_ref[...], w_ref[...], preferred_element_type=jnp.float32)
        + b_ref[...]
    )
    col = lax.broadcasted_iota(jnp.int32, (rb, D1P - D1), 1)
    pad = jnp.where(col == 0, 1.0, 0.0)
    o_ref[...] = jnp.concatenate([feat, pad], axis=1)


def _edge_proj_body(ea_ref, we1_ref, we2_ref, o1_ref, o2_ref):
    dn = (((0,), (0,)), ((), ()))
    ea_t = ea_ref[...]
    o1_ref[...] = lax.dot_general(ea_t, we1_ref[...], dn,
                                  preferred_element_type=jnp.float32)
    o2_ref[...] = lax.dot_general(ea_t, we2_ref[...], dn,
                                  preferred_element_type=jnp.float32)


def _norm_lin2_body(acc_ref, bias1_ref, w2_ref, b2_ref, o_ref):
    rb = o_ref.shape[0]
    acc = acc_ref[0] + acc_ref[1]
    den = acc[:, D1] + 1e-16
    lat = acc[:, :D1] / den[:, None] + bias1_ref[...]
    feat = (
        jnp.dot(lat, w2_ref[...], preferred_element_type=jnp.float32)
        + b2_ref[...]
    )
    col = lax.broadcasted_iota(jnp.int32, (rb, D2P - D2), 1)
    pad = jnp.where(col == 0, 1.0, 0.0)
    o_ref[...] = jnp.concatenate([feat, pad], axis=1)


def _norm2_body(acc_ref, bias2_ref, o_ref):
    acc = acc_ref[0] + acc_ref[1]
    den = acc[:, D2] + 1e-16
    o_ref[...] = acc[:, :D2] / den[:, None] + bias2_ref[...]


_MESH = plsc.VectorSubcoreMesh(core_axis_name="c", subcore_axis_name="s")


def _sc_edge_body(dp, dfull, ch, xl_hbm, src_hbm, dst_hbm, ef_hbm, att_hbm,
                  acc_out,
                  acc_sh, srcv, dstv, sbuf, ab, bb, efb, ob, attv,
                  sem_i, sem_a, sem_b, sem_e, sem_s):
    cid = lax.axis_index("c")
    sid = lax.axis_index("s")
    wid = sid * NC + cid
    nj = dfull // L
    njp = dp // L
    nchunks = E // ch
    total = nchunks // NW
    assert total * NW == nchunks

    zero16 = jnp.zeros((L,), jnp.float32)

    def _zrow(i, carry):
        for j in range(njp):
            ob[i, pl.ds(L * j, L)] = zero16
        return carry

    lax.fori_loop(0, ch, _zrow, 0)

    base_row = sid * 624
    _sizes, _offs, _rem = [], [], 624
    while _rem:
        _offs.append(624 - _rem)
        _sizes.append(min(_rem, ch))
        _rem -= _sizes[-1]
    for sz, off in zip(_sizes, _offs):
        pltpu.sync_copy(ob.at[pl.ds(0, sz)],
                        acc_sh.at[pl.ds(base_row + off, sz)])

    @pl.when(sid == NS - 1)
    def _():
        pltpu.sync_copy(ob.at[pl.ds(0, 16)], acc_sh.at[pl.ds(624 * NS, 16)])

    plsc.subcore_barrier()

    pltpu.sync_copy(att_hbm, attv)
    att_regs = [attv[pl.ds(L * j, L)] for j in range(nj)]
    lane = lax.iota(jnp.int32, L)

    def _ebase(i):
        return (wid + NW * i) * ch

    def _idx_start(p, i):
        eb = _ebase(i)
        pltpu.async_copy(src_hbm.at[pl.ds(eb, ch)], srcv[p], sem_i[p])
        pltpu.async_copy(dst_hbm.at[pl.ds(eb, ch)], dstv[p], sem_i[p])

    def _idx_wait(p):
        pltpu.make_async_copy(src_hbm.at[pl.ds(0, ch)], srcv[p],
                              sem_i[p]).wait()
        pltpu.make_async_copy(dst_hbm.at[pl.ds(0, ch)], dstv[p],
                              sem_i[p]).wait()

    def _gather_start(p, i):
        eb = _ebase(i)
        pltpu.async_copy(xl_hbm.at[srcv[p]], ab[p], sem_a[p])
        pltpu.async_copy(xl_hbm.at[dstv[p]], bb[p], sem_b[p])
        if dfull == D1:
            pltpu.async_copy(ef_hbm.at[pl.ds(eb, ch)], efb[p], sem_e[p])
        else:
            pltpu.async_copy(ef_hbm.at[pl.ds(eb, ch), pl.ds(0, dfull)],
                             efb[p], sem_e[p])

    def _gather_wait(p):
        pltpu.make_async_copy(xl_hbm.at[srcv[p]], ab[p], sem_a[p]).wait()
        pltpu.make_async_copy(xl_hbm.at[dstv[p]], bb[p], sem_b[p]).wait()
        if dfull == D1:
            pltpu.make_async_copy(ef_hbm.at[pl.ds(0, ch)], efb[p],
                                  sem_e[p]).wait()
        else:
            pltpu.make_async_copy(ef_hbm.at[pl.ds(0, ch), pl.ds(0, dfull)],
                                  efb[p], sem_e[p]).wait()

    def _scatter_wait():
        pltpu.make_async_copy(ob, acc_sh.at[sbuf], sem_s).wait()

    def _compute(p):
        for off in range(0, ch - L + 1, L):
            sbuf[pl.ds(off, L)] = dstv[p][pl.ds(off, L)]
        if ch % L:
            off = ch - L
            sbuf[pl.ds(off, L)] = dstv[p][pl.ds(off, L)]

        def _edge(e, cc):
            s = zero16
            a_regs = []
            for j in range(nj):
                sl = pl.ds(L * j, L)
                a = ab[p][e, sl]
                a_regs.append(a)
                v = a + bb[p][e, sl] + efb[p][e, sl]
                lv = jnp.where(v > 0, v, 0.2 * v)
                s = s + lv * att_regs[j]
            wv = jnp.exp(jnp.broadcast_to(jnp.sum(s), (L,)))
            for j in range(nj):
                ob[e, pl.ds(L * j, L)] = a_regs[j] * wv
            ob[e, pl.ds(nj * L, L)] = jnp.where(lane == 0, wv, 0.0)
            return cc

        lax.fori_loop(0, ch, _edge, 0)

    def _scatter_start():
        pltpu.async_copy(ob, acc_sh.at[sbuf], sem_s, add=True)

    _idx_start(0, 0)
    _idx_start(1, 1)
    _idx_wait(0)
    _gather_start(0, 0)

    def _iter(i, p):
        @pl.when(i + 1 < total)
        def _():
            _idx_wait(1 - p)
            _gather_start(1 - p, i + 1)

        _gather_wait(p)

        @pl.when(i + 2 < total)
        def _():
            _idx_start(p, i + 2)

        @pl.when(i >= 1)
        def _():
            _scatter_wait()

        _compute(p)
        _scatter_start()

    def _pair(t, carry):
        _iter(2 * t, 0)
        _iter(2 * t + 1, 1)
        return carry

    lax.fori_loop(0, total // 2, _pair, 0)
    if total % 2:
        _iter(total - 1, 0)
    _scatter_wait()

    plsc.subcore_barrier()

    for sz, off in zip(_sizes, _offs):
        sl = pl.ds(base_row + off, sz)
        pltpu.sync_copy(acc_sh.at[sl], acc_out.at[cid, sl])

    @pl.when(sid == NS - 1)
    def _():
        sl = pl.ds(624 * NS, 16)
        pltpu.sync_copy(acc_sh.at[sl], acc_out.at[cid, sl])


def _make_sc_edge(dp, dfull, ch):
    return functools.partial(
        pl.kernel,
        out_type=jax.ShapeDtypeStruct((NC, N, dp), jnp.float32),
        mesh=_MESH,
        scratch_types=[
            pltpu.VMEM_SHARED((N, dp), jnp.float32),
            [pltpu.VMEM((ch,), jnp.int32)] * 2,
            [pltpu.VMEM((ch,), jnp.int32)] * 2,
            pltpu.VMEM((ch,), jnp.int32),
            [pltpu.VMEM((ch, dp), jnp.float32)] * 2,
            [pltpu.VMEM((ch, dp), jnp.float32)] * 2,
            [pltpu.VMEM((ch, dfull), jnp.float32)] * 2,
            pltpu.VMEM((ch, dp), jnp.float32),
            pltpu.VMEM((dfull,), jnp.float32),
            [pltpu.SemaphoreType.DMA] * 2,
            [pltpu.SemaphoreType.DMA] * 2,
            [pltpu.SemaphoreType.DMA] * 2,
            [pltpu.SemaphoreType.DMA] * 2,
            pltpu.SemaphoreType.DMA,
        ],
        compiler_params=pltpu.CompilerParams(use_tc_tiling_on_sc=False,
                                             needs_layout_passes=False),
    )(functools.partial(_sc_edge_body, dp, dfull, ch))


_sc_edge1 = _make_sc_edge(D1P, D1, 40)
_sc_edge2 = _make_sc_edge(D2P, D2, 80)


def kernel(x, edge_index, edge_attr, W1, b1, We1, att1, bias1,
           W2, b2, We2, att2, bias2):
    src = edge_index[0]
    dst = edge_index[1]

    RB = 400
    xl1 = pl.pallas_call(
        _lin1_body,
        grid=(N // RB,),
        in_specs=[
            pl.BlockSpec((RB, D1), lambda i: (i, 0)),
            pl.BlockSpec((D1, D1), lambda i: (0, 0)),
            pl.BlockSpec((D1,), lambda i: (0,)),
        ],
        out_specs=pl.BlockSpec((RB, D1P), lambda i: (i, 0)),
        out_shape=jax.ShapeDtypeStruct((N, D1P), jnp.float32),
    )(x, W1, b1)

    EB = 6400
    We2p = jnp.pad(We2, ((0, 0), (0, D1 - D2)))
    ef1, ef2 = pl.pallas_call(
        _edge_proj_body,
        grid=(E // EB,),
        in_specs=[
            pl.BlockSpec((D2, EB), lambda i: (0, i)),
            pl.BlockSpec((D2, D1), lambda i: (0, 0)),
            pl.BlockSpec((D2, D1), lambda i: (0, 0)),
        ],
        out_specs=[
            pl.BlockSpec((EB, D1), lambda i: (i, 0)),
            pl.BlockSpec((EB, D1), lambda i: (i, 0)),
        ],
        out_shape=[
            jax.ShapeDtypeStruct((E, D1), jnp.float32),
            jax.ShapeDtypeStruct((E, D1), jnp.float32),
        ],
    )(edge_attr.T, We1, We2p)

    acc1 = _sc_edge1(xl1, src, dst, ef1, att1)

    xl2 = pl.pallas_call(
        _norm_lin2_body,
        grid=(N // RB,),
        in_specs=[
            pl.BlockSpec((NC, RB, D1P), lambda i: (0, i, 0)),
            pl.BlockSpec((D1,), lambda i: (0,)),
            pl.BlockSpec((D1, D2), lambda i: (0, 0)),
            pl.BlockSpec((D2,), lambda i: (0,)),
        ],
        out_specs=pl.BlockSpec((RB, D2P), lambda i: (i, 0)),
        out_shape=jax.ShapeDtypeStruct((N, D2P), jnp.float32),
    )(acc1, bias1, W2, b2)

    acc2 = _sc_edge2(xl2, src, dst, ef2, att2)

    action_logits = pl.pallas_call(
        _norm2_body,
        grid=(N // RB,),
        in_specs=[
            pl.BlockSpec((NC, RB, D2P), lambda i: (0, i, 0)),
            pl.BlockSpec((D2,), lambda i: (0,)),
        ],
        out_specs=pl.BlockSpec((RB, D2), lambda i: (i, 0)),
        out_shape=jax.ShapeDtypeStruct((N, D2), jnp.float32),
    )(acc2, bias2)

    flat = action_logits.reshape(-1)
    skey = jax.random.key(42)
    idx = jax.random.categorical(skey, flat)
    log_prob = jax.nn.log_softmax(flat)[idx]
    sel_node, sel_action = jnp.unravel_index(idx, action_logits.shape)
    return (sel_node, sel_action, log_prob)


# speedup vs baseline: 13.5791x; 1.0101x over previous
import functools

import jax
import jax.numpy as jnp
from jax import lax
from jax.experimental import pallas as pl
from jax.experimental.pallas import tpu as pltpu
from jax.experimental.pallas import tpu_sc as plsc

N = 10000
E = 320000
D1 = 128
D2 = 16
D1P = D1 + 16
D2P = D2 + 16
NC, NS, L = 2, 16, 16
NW = NC * NS


def _lin1_body(x_ref, w_ref, b_ref, o_ref):
    rb = x_ref.shape[0]
    feat = (
        jnp.dot(x_ref[...], w_ref[...], preferred_element_type=jnp.float32)
        + b_ref[...]
    )
    col = lax.broadcasted_iota(jnp.int32, (rb, D1P - D1), 1)
    pad = jnp.where(col == 0, 1.0, 0.0)
    o_ref[...] = jnp.concatenate([feat, pad], axis=1)


def _edge_proj_body(ea_ref, we1_ref, we2_ref, o1_ref, o2_ref):
    dn = (((0,), (0,)), ((), ()))
    ea_t = ea_ref[...]
    o1_ref[...] = lax.dot_general(ea_t, we1_ref[...], dn,
                                  preferred_element_type=jnp.float32)
    o2_ref[...] = lax.dot_general(ea_t, we2_ref[...], dn,
                                  preferred_element_type=jnp.float32)


def _norm_lin2_body(acc_ref, bias1_ref, w2_ref, b2_ref, o_ref):
    rb = o_ref.shape[0]
    acc = acc_ref[0] + acc_ref[1]
    den = acc[:, D1] + 1e-16
    lat = acc[:, :D1] / den[:, None] + bias1_ref[...]
    feat = (
        jnp.dot(lat, w2_ref[...], preferred_element_type=jnp.float32)
        + b2_ref[...]
    )
    col = lax.broadcasted_iota(jnp.int32, (rb, D2P - D2), 1)
    pad = jnp.where(col == 0, 1.0, 0.0)
    o_ref[...] = jnp.concatenate([feat, pad], axis=1)


def _norm2_body(acc_ref, bias2_ref, o_ref):
    acc = acc_ref[0] + acc_ref[1]
    den = acc[:, D2] + 1e-16
    o_ref[...] = acc[:, :D2] / den[:, None] + bias2_ref[...]


_MESH = plsc.VectorSubcoreMesh(core_axis_name="c", subcore_axis_name="s")


def _sc_edge_body(dp, dfull, ch, xl_hbm, ei_hbm, ef_hbm, att_hbm,
                  acc_out,
                  acc_sh, idx2, sbuf, ab, bb, efb, ob, attv,
                  sem_i, sem_a, sem_b, sem_e, sem_s):
    cid = lax.axis_index("c")
    sid = lax.axis_index("s")
    wid = sid * NC + cid
    nj = dfull // L
    njp = dp // L
    nchunks = E // ch
    total = nchunks // NW
    assert total * NW == nchunks

    zero16 = jnp.zeros((L,), jnp.float32)

    def _zrow(i, carry):
        for j in range(njp):
            ob[i, pl.ds(L * j, L)] = zero16
        return carry

    lax.fori_loop(0, ch, _zrow, 0)

    base_row = sid * 624
    _sizes, _offs, _rem = [], [], 624
    while _rem:
        _offs.append(624 - _rem)
        _sizes.append(min(_rem, ch))
        _rem -= _sizes[-1]
    for sz, off in zip(_sizes, _offs):
        pltpu.sync_copy(ob.at[pl.ds(0, sz)],
                        acc_sh.at[pl.ds(base_row + off, sz)])

    @pl.when(sid == NS - 1)
    def _():
        pltpu.sync_copy(ob.at[pl.ds(0, 16)], acc_sh.at[pl.ds(624 * NS, 16)])

    plsc.subcore_barrier()

    pltpu.sync_copy(att_hbm, attv)
    att_regs = [attv[pl.ds(L * j, L)] for j in range(nj)]
    lane = lax.iota(jnp.int32, L)

    def _ebase(i):
        return (wid + NW * i) * ch

    def _idx_start(p, i):
        eb = _ebase(i)
        pltpu.async_copy(ei_hbm.at[:, pl.ds(eb, ch)], idx2[p], sem_i[p])

    def _idx_wait(p):
        pltpu.make_async_copy(ei_hbm.at[:, pl.ds(0, ch)], idx2[p],
                              sem_i[p]).wait()

    def _gather_start(p, i):
        eb = _ebase(i)
        pltpu.async_copy(xl_hbm.at[idx2[p].at[0]], ab[p], sem_a[p])
        pltpu.async_copy(xl_hbm.at[idx2[p].at[1]], bb[p], sem_b[p])
        if dfull == D1:
            pltpu.async_copy(ef_hbm.at[pl.ds(eb, ch)], efb[p], sem_e[p])
        else:
            pltpu.async_copy(ef_hbm.at[pl.ds(eb, ch), pl.ds(0, dfull)],
                             efb[p], sem_e[p])

    def _gather_wait(p):
        pltpu.make_async_copy(xl_hbm.at[idx2[p].at[0]], ab[p],
                              sem_a[p]).wait()
        pltpu.make_async_copy(xl_hbm.at[idx2[p].at[1]], bb[p],
                              sem_b[p]).wait()
        if dfull == D1:
            pltpu.make_async_copy(ef_hbm.at[pl.ds(0, ch)], efb[p],
                                  sem_e[p]).wait()
        else:
            pltpu.make_async_copy(ef_hbm.at[pl.ds(0, ch), pl.ds(0, dfull)],
                                  efb[p], sem_e[p]).wait()

    def _scatter_wait():
        pltpu.make_async_copy(ob, acc_sh.at[sbuf], sem_s).wait()

    def _compute(p):
        for off in range(0, ch - L + 1, L):
            sbuf[pl.ds(off, L)] = idx2[p][1, pl.ds(off, L)]
        if ch % L:
            off = ch - L
            sbuf[pl.ds(off, L)] = idx2[p][1, pl.ds(off, L)]

        def _edge(e, cc):
            s = zero16
            a_regs = []
            for j in range(nj):
                sl = pl.ds(L * j, L)
                a = ab[p][e, sl]
                a_regs.append(a)
                v = a + bb[p][e, sl] + efb[p][e, sl]
                lv = jnp.where(v > 0, v, 0.2 * v)
                s = s + lv * att_regs[j]
            wv = jnp.exp(jnp.broadcast_to(jnp.sum(s), (L,)))
            for j in range(nj):
                ob[e, pl.ds(L * j, L)] = a_regs[j] * wv
            ob[e, pl.ds(nj * L, L)] = jnp.where(lane == 0, wv, 0.0)
            return cc

        lax.fori_loop(0, ch, _edge, 0)

    def _scatter_start():
        pltpu.async_copy(ob, acc_sh.at[sbuf], sem_s, add=True)

    _idx_start(0, 0)
    _idx_start(1, 1)
    _idx_wait(0)
    _gather_start(0, 0)

    def _iter(i, p):
        @pl.when(i + 1 < total)
        def _():
            _idx_wait(1 - p)
            _gather_start(1 - p, i + 1)

        _gather_wait(p)

        @pl.when(i + 2 < total)
        def _():
            _idx_start(p, i + 2)

        @pl.when(i >= 1)
        def _():
            _scatter_wait()

        _compute(p)
        _scatter_start()

    def _pair(t, carry):
        _iter(2 * t, 0)
        _iter(2 * t + 1, 1)
        return carry

    lax.fori_loop(0, total // 2, _pair, 0)
    if total % 2:
        _iter(total - 1, 0)
    _scatter_wait()

    plsc.subcore_barrier()

    for sz, off in zip(_sizes, _offs):
        sl = pl.ds(base_row + off, sz)
        pltpu.sync_copy(acc_sh.at[sl], acc_out.at[cid, sl])

    @pl.when(sid == NS - 1)
    def _():
        sl = pl.ds(624 * NS, 16)
        pltpu.sync_copy(acc_sh.at[sl], acc_out.at[cid, sl])


def _make_sc_edge(dp, dfull, ch):
    return functools.partial(
        pl.kernel,
        out_type=jax.ShapeDtypeStruct((NC, N, dp), jnp.float32),
        mesh=_MESH,
        scratch_types=[
            pltpu.VMEM_SHARED((N, dp), jnp.float32),
            [pltpu.VMEM((2, ch), jnp.int32)] * 2,
            pltpu.VMEM((ch,), jnp.int32),
            [pltpu.VMEM((ch, dp), jnp.float32)] * 2,
            [pltpu.VMEM((ch, dp), jnp.float32)] * 2,
            [pltpu.VMEM((ch, dfull), jnp.float32)] * 2,
            pltpu.VMEM((ch, dp), jnp.float32),
            pltpu.VMEM((dfull,), jnp.float32),
            [pltpu.SemaphoreType.DMA] * 2,
            [pltpu.SemaphoreType.DMA] * 2,
            [pltpu.SemaphoreType.DMA] * 2,
            [pltpu.SemaphoreType.DMA] * 2,
            pltpu.SemaphoreType.DMA,
        ],
        compiler_params=pltpu.CompilerParams(use_tc_tiling_on_sc=False,
                                             needs_layout_passes=False),
    )(functools.partial(_sc_edge_body, dp, dfull, ch))


_sc_edge1 = _make_sc_edge(D1P, D1, 40)
_sc_edge2 = _make_sc_edge(D2P, D2, 80)


def kernel(x, edge_index, edge_attr, W1, b1, We1, att1, bias1,
           W2, b2, We2, att2, bias2):
    RB = 400
    xl1 = pl.pallas_call(
        _lin1_body,
        grid=(N // RB,),
        in_specs=[
            pl.BlockSpec((RB, D1), lambda i: (i, 0)),
            pl.BlockSpec((D1, D1), lambda i: (0, 0)),
            pl.BlockSpec((D1,), lambda i: (0,)),
        ],
        out_specs=pl.BlockSpec((RB, D1P), lambda i: (i, 0)),
        out_shape=jax.ShapeDtypeStruct((N, D1P), jnp.float32),
    )(x, W1, b1)

    EB = 6400
    We2p = jnp.pad(We2, ((0, 0), (0, D1 - D2)))
    ef1, ef2 = pl.pallas_call(
        _edge_proj_body,
        grid=(E // EB,),
        in_specs=[
            pl.BlockSpec((D2, EB), lambda i: (0, i)),
            pl.BlockSpec((D2, D1), lambda i: (0, 0)),
            pl.BlockSpec((D2, D1), lambda i: (0, 0)),
        ],
        out_specs=[
            pl.BlockSpec((EB, D1), lambda i: (i, 0)),
            pl.BlockSpec((EB, D1), lambda i: (i, 0)),
        ],
        out_shape=[
            jax.ShapeDtypeStruct((E, D1), jnp.float32),
            jax.ShapeDtypeStruct((E, D1), jnp.float32),
        ],
    )(edge_attr.T, We1, We2p)

    acc1 = _sc_edge1(xl1, edge_index, ef1, att1)

    xl2 = pl.pallas_call(
        _norm_lin2_body,
        grid=(N // RB,),
        in_specs=[
            pl.BlockSpec((NC, RB, D1P), lambda i: (0, i, 0)),
            pl.BlockSpec((D1,), lambda i: (0,)),
            pl.BlockSpec((D1, D2), lambda i: (0, 0)),
            pl.BlockSpec((D2,), lambda i: (0,)),
        ],
        out_specs=pl.BlockSpec((RB, D2P), lambda i: (i, 0)),
        out_shape=jax.ShapeDtypeStruct((N, D2P), jnp.float32),
    )(acc1, bias1, W2, b2)

    acc2 = _sc_edge2(xl2, edge_index, ef2, att2)

    action_logits = pl.pallas_call(
        _norm2_body,
        grid=(N // RB,),
        in_specs=[
            pl.BlockSpec((NC, RB, D2P), lambda i: (0, i, 0)),
            pl.BlockSpec((D2,), lambda i: (0,)),
        ],
        out_specs=pl.BlockSpec((RB, D2), lambda i: (i, 0)),
        out_shape=jax.ShapeDtypeStruct((N, D2), jnp.float32),
    )(acc2, bias2)

    flat = action_logits.reshape(-1)
    skey = jax.random.key(42)
    idx = jax.random.categorical(skey, flat)
    log_prob = jax.nn.log_softmax(flat)[idx]
    sel_node, sel_action = jnp.unravel_index(idx, action_logits.shape)
    return (sel_node, sel_action, log_prob)
